# baseline (speedup 1.0000x reference)
import functools

import jax
import jax.numpy as jnp
from jax import lax
from jax.experimental import pallas as pl
from jax.experimental.pallas import tpu as pltpu
from jax.experimental.pallas import tpu_sc as plsc

_INFO = plsc.get_sparse_core_info()
_NC = _INFO.num_cores
_NS = _INFO.num_subcores
_NW = _NC * _NS

_NBUF = 8


def _make_gather(n_tok: int, d: int, chunk: int):
    b_per_w = n_tok // _NW
    n_chunks = b_per_w // chunk
    assert n_chunks % _NBUF == 0 and n_chunks >= 2 * _NBUF
    mesh = plsc.VectorSubcoreMesh(core_axis_name="c", subcore_axis_name="s")

    @functools.partial(
        pl.kernel,
        mesh=mesh,
        out_type=jax.ShapeDtypeStruct((n_tok, d), jnp.float32),
        scratch_types=[
            pltpu.VMEM((n_chunks, chunk), jnp.int32),
        ] + [pltpu.VMEM((chunk, d), jnp.float32)] * _NBUF
          + [pltpu.SemaphoreType.DMA] * (2 * _NBUF),
    )
    def gather_kernel(table_hbm, idx_hbm, out_hbm, idx_v, *rest):
        bufs = rest[:_NBUF]
        gsems = rest[_NBUF:2 * _NBUF]
        wsems = rest[2 * _NBUF:]

        wid = lax.axis_index("s") * _NC + lax.axis_index("c")
        base = wid * b_per_w
        pltpu.sync_copy(idx_hbm.at[wid], idx_v)

        def out_rows(g):
            return out_hbm.at[pl.ds(base + g * chunk, chunk)]

        def start_gather(g, b):
            pltpu.async_copy(table_hbm.at[idx_v.at[g]], bufs[b], gsems[b])

        for b in range(_NBUF - 1):
            start_gather(b, b)

        def ring_body(t, carry):
            for b in range(_NBUF):
                g = _NBUF * t + b
                bn = (b + _NBUF - 1) % _NBUF
                pltpu.make_async_copy(
                    table_hbm.at[idx_v.at[g]], bufs[b], gsems[b]).wait()

                @pl.when((g >= 1) & (g + _NBUF - 1 < n_chunks))
                def _():
                    pltpu.make_async_copy(
                        bufs[bn], out_rows(g - 1), wsems[bn]).wait()

                @pl.when(g + _NBUF - 1 < n_chunks)
                def _():
                    start_gather(g + _NBUF - 1, bn)

                pltpu.async_copy(bufs[b], out_rows(g), wsems[b])
            return carry

        lax.fori_loop(0, n_chunks // _NBUF, ring_body, 0)

        for b in range(_NBUF):
            g = n_chunks - _NBUF + b
            pltpu.make_async_copy(bufs[b], out_rows(g), wsems[b]).wait()

    return gather_kernel


def kernel(idx, embedding):
    b, s = idx.shape
    v, d = embedding.shape
    n_tok = b * s
    chunk = 1
    idx32 = idx.reshape(_NW, (n_tok // _NW) // chunk, chunk).astype(jnp.int32)
    out = _make_gather(n_tok, d, chunk)(embedding, idx32)
    return out.reshape(b, s, d)


# speedup vs baseline: 1.0266x; 1.0266x over previous
import functools

import jax
import jax.numpy as jnp
from jax import lax
from jax.experimental import pallas as pl
from jax.experimental.pallas import tpu as pltpu
from jax.experimental.pallas import tpu_sc as plsc

_INFO = plsc.get_sparse_core_info()
_NC = _INFO.num_cores
_NS = _INFO.num_subcores
_NW = _NC * _NS

_NBUF = 3
_CHUNK = 4


def _make_gather(n_tok: int, d: int, chunk: int, nbuf: int):
    b_per_w = n_tok // _NW
    n_chunks = b_per_w // chunk
    assert n_chunks >= 2 * nbuf
    mesh = plsc.VectorSubcoreMesh(core_axis_name="c", subcore_axis_name="s")

    @functools.partial(
        pl.kernel,
        mesh=mesh,
        out_type=jax.ShapeDtypeStruct((n_tok, d), jnp.float32),
        scratch_types=[
            pltpu.VMEM((n_chunks, chunk), jnp.int32),
        ] + [pltpu.VMEM((chunk, d), jnp.float32)] * nbuf
          + [pltpu.SemaphoreType.DMA] * (2 * nbuf),
    )
    def gather_kernel(table_hbm, idx_hbm, out_hbm, idx_v, *rest):
        bufs = rest[:nbuf]
        gsems = rest[nbuf:2 * nbuf]
        wsems = rest[2 * nbuf:]

        wid = lax.axis_index("s") * _NC + lax.axis_index("c")
        base = wid * b_per_w
        pltpu.sync_copy(idx_hbm.at[wid], idx_v)

        def out_rows(g):
            return out_hbm.at[pl.ds(base + g * chunk, chunk)]

        def start_gather(g, b):
            pltpu.async_copy(table_hbm.at[idx_v.at[g]], bufs[b], gsems[b])

        def visit(g, b, static):
            bn = (b + nbuf - 1) % nbuf
            pltpu.make_async_copy(
                table_hbm.at[idx_v.at[g]], bufs[b], gsems[b]).wait()

            def recycle_wait():
                pltpu.make_async_copy(
                    bufs[bn], out_rows(g - 1), wsems[bn]).wait()

            def next_gather():
                start_gather(g + nbuf - 1, bn)

            if static:
                if g >= 1 and g + nbuf - 1 < n_chunks:
                    recycle_wait()
                if g + nbuf - 1 < n_chunks:
                    next_gather()
            else:
                pl.when((g >= 1) & (g + nbuf - 1 < n_chunks))(recycle_wait)
                pl.when(g + nbuf - 1 < n_chunks)(next_gather)

            pltpu.async_copy(bufs[b], out_rows(g), wsems[b])

        for b in range(nbuf - 1):
            start_gather(b, b)

        n_full = (n_chunks // nbuf) * nbuf

        def ring_body(t, carry):
            for b in range(nbuf):
                visit(nbuf * t + b, b, static=False)
            return carry

        lax.fori_loop(0, n_full // nbuf, ring_body, 0)

        for g in range(n_full, n_chunks):
            visit(g, g % nbuf, static=True)

        for g in range(n_chunks - nbuf, n_chunks):
            b = g % nbuf
            pltpu.make_async_copy(bufs[b], out_rows(g), wsems[b]).wait()

    return gather_kernel


def kernel(idx, embedding):
    b, s = idx.shape
    v, d = embedding.shape
    n_tok = b * s
    idx32 = idx.reshape(_NW, (n_tok // _NW) // _CHUNK, _CHUNK).astype(jnp.int32)
    out = _make_gather(n_tok, d, _CHUNK, _NBUF)(embedding, idx32)
    return out.reshape(b, s, d)
